# baseline (speedup 1.0000x reference)
import functools

import jax
import jax.numpy as jnp
from jax import lax
from jax.experimental import pallas as pl
from jax.experimental.pallas import tpu as pltpu
from jax.experimental.pallas import tpu_sc as plsc

BATCH = 16384
D = 64
L = 16
NC, NS = 2, 16
NW = NC * NS
BPW = BATCH // NW
CHUNK = 128
NCHUNK = BPW // CHUNK
GROUPS = BPW // L

_mesh = plsc.VectorSubcoreMesh(core_axis_name="c", subcore_axis_name="s")


@functools.partial(
    pl.kernel,
    mesh=_mesh,
    out_type=jax.ShapeDtypeStruct((BATCH,), jnp.float32),
    compiler_params=pltpu.CompilerParams(needs_layout_passes=False,
                                         use_tc_tiling_on_sc=False),
    scratch_types=[
        pltpu.VMEM((BPW,), jnp.int32),
        pltpu.VMEM((BPW,), jnp.int32),
        pltpu.VMEM((2 * BPW,), jnp.float32),
        pltpu.VMEM((2 * BPW,), jnp.float32),
        pltpu.VMEM((BPW,), jnp.float32),
        pltpu.SemaphoreType.DMA,
        pltpu.SemaphoreType.DMA,
        pltpu.SemaphoreType.DMA,
        pltpu.SemaphoreType.DMA,
    ],
)
def _bi_encoder(uidx_hbm, bidx_hbm, utab_hbm, btab_hbm, out_hbm,
                uidx_v, bidx_v, ubuf, bbuf, acc_v,
                sem_u0, sem_u1, sem_b0, sem_b1):
    wid = lax.axis_index("s") * NC + lax.axis_index("c")
    base = wid * BPW

    pltpu.sync_copy(uidx_hbm.at[pl.ds(base, BPW)], uidx_v)
    pltpu.sync_copy(bidx_hbm.at[pl.ds(base, BPW)], bidx_v)

    def zbody(g, carry):
        acc_v[pl.ds(g * L, L)] = jnp.zeros((L,), jnp.float32)
        return carry

    lax.fori_loop(0, GROUPS, zbody, 0)

    sems_u = (sem_u0, sem_u1)
    sems_b = (sem_b0, sem_b1)

    def enqueue(d, slot):
        for ch in range(NCHUNK):
            isl = pl.ds(ch * CHUNK, CHUNK)
            dsl = pl.ds(slot * BPW + ch * CHUNK, CHUNK)
            pltpu.async_copy(utab_hbm.at[d].at[uidx_v.at[isl]], ubuf.at[dsl],
                             sems_u[slot])
            pltpu.async_copy(btab_hbm.at[d].at[bidx_v.at[isl]], bbuf.at[dsl],
                             sems_b[slot])

    def wait(d, slot):
        dsl = pl.ds(slot * BPW, BPW)
        ssl = pl.ds(0, BPW)
        pltpu.make_async_copy(utab_hbm.at[d].at[ssl], ubuf.at[dsl],
                              sems_u[slot]).wait()
        pltpu.make_async_copy(btab_hbm.at[d].at[ssl], bbuf.at[dsl],
                              sems_b[slot]).wait()

    def accum(slot):
        def abody(g, carry):
            sl = pl.ds(g * L, L)
            bsl = pl.ds(slot * BPW + g * L, L)
            acc_v[sl] = acc_v[sl] + ubuf[bsl] * bbuf[bsl]
            return carry

        lax.fori_loop(0, GROUPS, abody, 0, unroll=4)

    enqueue(0, 0)

    def dbody(k, carry):
        d0 = 2 * k
        enqueue(d0 + 1, 1)
        wait(d0, 0)
        accum(0)

        @pl.when(k + 1 < D // 2)
        def _():
            enqueue(d0 + 2, 0)

        wait(d0 + 1, 1)
        accum(1)
        return carry

    lax.fori_loop(0, D // 2, dbody, 0)

    def sbody(g, carry):
        sl = pl.ds(g * L, L)
        acc_v[sl] = 1.0 / (1.0 + jnp.exp(-acc_v[sl]))
        return carry

    lax.fori_loop(0, GROUPS, sbody, 0)

    pltpu.sync_copy(acc_v, out_hbm.at[pl.ds(base, BPW)])


def kernel(user_indices, book_indices, user_table, book_table):
    return _bi_encoder(user_indices.astype(jnp.int32),
                       book_indices.astype(jnp.int32),
                       user_table.T, book_table.T)


# speedup vs baseline: 8.9156x; 8.9156x over previous
import functools

import jax
import jax.numpy as jnp
from jax import lax
from jax.experimental import pallas as pl
from jax.experimental.pallas import tpu as pltpu
from jax.experimental.pallas import tpu_sc as plsc

BATCH = 16384
D = 64
L = 16
NC, NS = 2, 16
NW = NC * NS
BPW = BATCH // NW
CHUNK = 128
NCHUNK = BPW // CHUNK
GROUPS = BPW // L

_mesh = plsc.VectorSubcoreMesh(core_axis_name="c", subcore_axis_name="s")


@functools.partial(
    pl.kernel,
    mesh=_mesh,
    out_type=jax.ShapeDtypeStruct((BATCH,), jnp.float32),
    compiler_params=pltpu.CompilerParams(needs_layout_passes=False,
                                         use_tc_tiling_on_sc=False),
    scratch_types=[
        pltpu.VMEM((BPW,), jnp.int32),
        pltpu.VMEM((BPW,), jnp.int32),
        pltpu.VMEM((BPW, D), jnp.float32),
        pltpu.VMEM((BPW, D), jnp.float32),
        pltpu.VMEM((BPW,), jnp.float32),
        pltpu.SemaphoreType.DMA,
        pltpu.SemaphoreType.DMA,
    ],
)
def _bi_encoder(uidx_hbm, bidx_hbm, utab_hbm, btab_hbm, out_hbm,
                uidx_v, bidx_v, urows_v, brows_v, out_v, sem_u, sem_b):
    wid = lax.axis_index("s") * NC + lax.axis_index("c")
    base = wid * BPW

    pltpu.sync_copy(uidx_hbm.at[pl.ds(base, BPW)], uidx_v)
    pltpu.sync_copy(bidx_hbm.at[pl.ds(base, BPW)], bidx_v)

    copies = []
    for c in range(NCHUNK):
        sl = pl.ds(c * CHUNK, CHUNK)
        copies.append(
            pltpu.async_copy(utab_hbm.at[uidx_v.at[sl]], urows_v.at[sl], sem_u))
        copies.append(
            pltpu.async_copy(btab_hbm.at[bidx_v.at[sl]], brows_v.at[sl], sem_b))
    for cp in copies:
        cp.wait()

    viota = lax.iota(jnp.int32, L)

    def gbody(g, carry):
        rows = g * L + viota

        def dbody(d, acc):
            cols = jnp.full((L,), d, jnp.int32)
            uu = plsc.load_gather(urows_v, [rows, cols])
            bb = plsc.load_gather(brows_v, [rows, cols])
            return acc + uu * bb

        acc = lax.fori_loop(0, D, dbody, jnp.zeros((L,), jnp.float32),
                            unroll=8)
        out_v[pl.ds(g * L, L)] = 1.0 / (1.0 + jnp.exp(-acc))
        return carry

    lax.fori_loop(0, GROUPS, gbody, 0)

    pltpu.sync_copy(out_v, out_hbm.at[pl.ds(base, BPW)])


def kernel(user_indices, book_indices, user_table, book_table):
    return _bi_encoder(user_indices.astype(jnp.int32),
                       book_indices.astype(jnp.int32),
                       user_table, book_table)
